# baseline (speedup 1.0000x reference)
import functools

import jax
import jax.numpy as jnp
from jax import lax
from jax.experimental import pallas as pl
from jax.experimental.pallas import tpu as pltpu
from jax.experimental.pallas import tpu_sc as plsc

_NC = 2
_NS = 16
_NW = _NC * _NS
_L = 16
_W = 4096
_TILE = 128
_ICHUNK = 128


def _detile_body(V, D, n_full, tail_base, tail_n,
                 tbl_a, tbl_b, tail_a, tail_b, out_a, out_b, sem):
    wid = lax.axis_index("s") * _NC + lax.axis_index("c")
    for tbl, tail, out in ((tbl_a, tail_a, out_a), (tbl_b, tail_b, out_b)):
        n_rounds = -(-(n_full + 1) // _NW)
        for k in range(n_rounds):
            c = wid + k * _NW

            @pl.when(c < n_full)
            def _full(tbl=tbl, out=out, c=c):
                start = pl.multiple_of(c * _W, _TILE)
                pltpu.sync_copy(tbl.at[:, pl.ds(start, _W)],
                                out.at[:, pl.ds(start, _W)])

            @pl.when(c == n_full)
            def _rem(tbl=tbl, out=out):
                start = pl.multiple_of(n_full * _W, _TILE)
                rem = tail_base - n_full * _W
                pltpu.sync_copy(tbl.at[:, pl.ds(start, rem)],
                                out.at[:, pl.ds(start, rem)])

        @pl.when(wid == _NW - 1)
        def _tail(tail=tail, out=out):
            pltpu.sync_copy(tail, out.at[:, pl.ds(tail_base, tail_n)])


def _gather_body(n_per_w, D, idx0_hbm, idx1_hbm, idx2_hbm, lin_in_hbm,
                 lin_w_hbm, emb_b_hbm, out_hbm,
                 idx0_v, idx1_v, idx2_v, e0_v, e1_v, w_v, b_v, out_v, sem):
    wid = lax.axis_index("s") * _NC + lax.axis_index("c")
    base = wid * n_per_w

    pltpu.sync_copy(idx0_hbm.at[pl.ds(base, n_per_w)], idx0_v)
    pltpu.sync_copy(idx1_hbm.at[pl.ds(base, n_per_w)], idx1_v)
    pltpu.sync_copy(idx2_hbm.at[pl.ds(base, n_per_w)], idx2_v)

    copies = []
    for k in range(0, n_per_w, _ICHUNK):
        sl = pl.ds(k, _ICHUNK)
        for d in range(D):
            copies.append(pltpu.async_copy(
                lin_in_hbm.at[d].at[idx0_v.at[sl]], e0_v.at[d].at[sl], sem))
            copies.append(pltpu.async_copy(
                lin_in_hbm.at[d].at[idx1_v.at[sl]], e1_v.at[d].at[sl], sem))
            copies.append(pltpu.async_copy(
                lin_w_hbm.at[d].at[idx2_v.at[sl]], w_v.at[d].at[sl], sem))
        copies.append(pltpu.async_copy(
            emb_b_hbm.at[idx2_v.at[sl]], b_v.at[sl], sem))
    for c in copies:
        c.wait()

    def group(g, _):
        sl = pl.ds(g * _L, _L)
        acc = jnp.zeros((_L,), jnp.float32)
        for d in range(D):
            acc = acc + (e0_v[d, sl] + e1_v[d, sl]) * w_v[d, sl]
        logit = acc * 0.5 + b_v[sl]
        out_v[sl] = 1.0 / (1.0 + jnp.exp(-logit))
        return _

    lax.fori_loop(0, n_per_w // _L, group, None)

    pltpu.sync_copy(out_v, out_hbm.at[pl.ds(base, n_per_w)])


def kernel(x, emb_in, emb_out_w, emb_out_b):
    B = x.shape[0]
    V, D = emb_in.shape
    tail_base = (V // _TILE) * _TILE
    tail_n = V - tail_base
    n_full = V // _W
    n_per_w = B // _NW

    mesh = plsc.VectorSubcoreMesh(core_axis_name="c", subcore_axis_name="s")

    detile = pl.kernel(
        functools.partial(_detile_body, V, D, n_full, tail_base, tail_n),
        out_type=(jax.ShapeDtypeStruct((D, V), jnp.float32),
                  jax.ShapeDtypeStruct((D, V), jnp.float32)),
        mesh=mesh,
        scratch_types=[pltpu.SemaphoreType.DMA],
        compiler_params=pltpu.CompilerParams(
            needs_layout_passes=False, use_tc_tiling_on_sc=True),
    )
    tail_in = emb_in[tail_base:].T
    tail_w = emb_out_w[tail_base:].T
    lin_in, lin_w = detile(emb_in.T, emb_out_w.T, tail_in, tail_w)

    gather = pl.kernel(
        functools.partial(_gather_body, n_per_w, D),
        out_type=jax.ShapeDtypeStruct((B,), jnp.float32),
        mesh=mesh,
        scratch_types=[
            pltpu.VMEM((n_per_w,), jnp.int32),
            pltpu.VMEM((n_per_w,), jnp.int32),
            pltpu.VMEM((n_per_w,), jnp.int32),
            pltpu.VMEM((D, n_per_w), jnp.float32),
            pltpu.VMEM((D, n_per_w), jnp.float32),
            pltpu.VMEM((D, n_per_w), jnp.float32),
            pltpu.VMEM((n_per_w,), jnp.float32),
            pltpu.VMEM((n_per_w,), jnp.float32),
            pltpu.SemaphoreType.DMA,
        ],
        compiler_params=pltpu.CompilerParams(
            needs_layout_passes=False, use_tc_tiling_on_sc=False),
    )
    out = gather(x[:, 0], x[:, 1], x[:, 2], lin_in, lin_w,
                 emb_out_b.reshape(V))
    return out.reshape(B, 1)


# speedup vs baseline: 2.4103x; 2.4103x over previous
import functools

import jax
import jax.numpy as jnp
from jax import lax
from jax.experimental import pallas as pl
from jax.experimental.pallas import tpu as pltpu
from jax.experimental.pallas import tpu_sc as plsc

_NC = 2
_NS = 16
_NW = _NC * _NS
_L = 16
_W = 2048
_TILE = 128
_ICHUNK = 128


def _detile_body(V, D, n_full, rem_w, tail_base, tail_n,
                 tbl_a, tbl_b, tail_a, tail_b, out_a, out_b,
                 buf0, buf1, tbuf, so0, so1):
    bufs = (buf0, buf1)
    sems = (so0, so1)
    wid = lax.axis_index("s") * _NC + lax.axis_index("c")
    n_rounds = -(-(n_full + 1) // _NW)

    def fire(tbl, out, c, rb):
        @pl.when(c < n_full)
        def _full():
            start = pl.multiple_of(c * _W, _TILE)
            pltpu.sync_copy(tbl.at[:, pl.ds(start, _W)], bufs[rb])
            pltpu.async_copy(bufs[rb], out.at[:, pl.ds(start, _W)], sems[rb])

        @pl.when(c == n_full)
        def _rem():
            start = pl.multiple_of(n_full * _W, _TILE)
            pltpu.sync_copy(tbl.at[:, pl.ds(start, rem_w)],
                            bufs[rb].at[:, pl.ds(0, rem_w)])
            pltpu.async_copy(bufs[rb].at[:, pl.ds(0, rem_w)],
                             out.at[:, pl.ds(start, rem_w)], sems[rb])

    def drain(out, c, rb):
        @pl.when(c < n_full)
        def _df():
            start = pl.multiple_of(c * _W, _TILE)
            pltpu.make_async_copy(
                bufs[rb], out.at[:, pl.ds(start, _W)], sems[rb]).wait()

        @pl.when(c == n_full)
        def _dr():
            start = pl.multiple_of(n_full * _W, _TILE)
            pltpu.make_async_copy(
                bufs[rb].at[:, pl.ds(0, rem_w)],
                out.at[:, pl.ds(start, rem_w)], sems[rb]).wait()

    for tbl, out in ((tbl_a, out_a), (tbl_b, out_b)):
        for k in range(n_rounds):
            if k >= 2:
                drain(out, wid + (k - 2) * _NW, k % 2)
            fire(tbl, out, wid + k * _NW, k % 2)
        for k in range(max(n_rounds - 2, 0), n_rounds):
            drain(out, wid + k * _NW, k % 2)

    @pl.when(wid == _NW - 1)
    def _tails():
        for tail, out in ((tail_a, out_a), (tail_b, out_b)):
            pltpu.sync_copy(tail, tbuf)
            pltpu.sync_copy(tbuf, out.at[:, pl.ds(tail_base, tail_n)])


def _gather_body(n_per_w, D, idx0_hbm, idx1_hbm, idx2_hbm, lin_in_hbm,
                 lin_w_hbm, emb_b_hbm, out_hbm,
                 idx0_v, idx1_v, idx2_v, e0_v, e1_v, w_v, b_v, out_v, sem):
    wid = lax.axis_index("s") * _NC + lax.axis_index("c")
    base = wid * n_per_w

    pltpu.sync_copy(idx0_hbm.at[pl.ds(base, n_per_w)], idx0_v)
    pltpu.sync_copy(idx1_hbm.at[pl.ds(base, n_per_w)], idx1_v)
    pltpu.sync_copy(idx2_hbm.at[pl.ds(base, n_per_w)], idx2_v)

    copies = []
    for k in range(0, n_per_w, _ICHUNK):
        sl = pl.ds(k, _ICHUNK)
        for d in range(D):
            copies.append(pltpu.async_copy(
                lin_in_hbm.at[d].at[idx0_v.at[sl]], e0_v.at[d].at[sl], sem))
            copies.append(pltpu.async_copy(
                lin_in_hbm.at[d].at[idx1_v.at[sl]], e1_v.at[d].at[sl], sem))
            copies.append(pltpu.async_copy(
                lin_w_hbm.at[d].at[idx2_v.at[sl]], w_v.at[d].at[sl], sem))
        copies.append(pltpu.async_copy(
            emb_b_hbm.at[idx2_v.at[sl]], b_v.at[sl], sem))
    for c in copies:
        c.wait()

    def group(g, _):
        sl = pl.ds(g * _L, _L)
        acc = jnp.zeros((_L,), jnp.float32)
        for d in range(D):
            acc = acc + (e0_v[d, sl] + e1_v[d, sl]) * w_v[d, sl]
        logit = acc * 0.5 + b_v[sl]
        out_v[sl] = 1.0 / (1.0 + jnp.exp(-logit))
        return _

    lax.fori_loop(0, n_per_w // _L, group, None)

    pltpu.sync_copy(out_v, out_hbm.at[pl.ds(base, n_per_w)])


def kernel(x, emb_in, emb_out_w, emb_out_b):
    B = x.shape[0]
    V, D = emb_in.shape
    tail_base = (V // _TILE) * _TILE
    tail_n = V - tail_base
    n_full = V // _W
    rem_w = tail_base - n_full * _W
    n_per_w = B // _NW

    mesh = plsc.VectorSubcoreMesh(core_axis_name="c", subcore_axis_name="s")

    detile = pl.kernel(
        functools.partial(_detile_body, V, D, n_full, rem_w, tail_base,
                          tail_n),
        out_type=(jax.ShapeDtypeStruct((D, V), jnp.float32),
                  jax.ShapeDtypeStruct((D, V), jnp.float32)),
        mesh=mesh,
        scratch_types=[
            pltpu.VMEM((D, _W), jnp.float32),
            pltpu.VMEM((D, _W), jnp.float32),
            pltpu.VMEM((D, 64), jnp.float32),
            pltpu.SemaphoreType.DMA,
            pltpu.SemaphoreType.DMA,
        ],
        compiler_params=pltpu.CompilerParams(
            needs_layout_passes=False, use_tc_tiling_on_sc=True),
    )
    tail_in = emb_in[tail_base:].T
    tail_w = emb_out_w[tail_base:].T
    lin_in, lin_w = detile(emb_in.T, emb_out_w.T, tail_in, tail_w)

    gather = pl.kernel(
        functools.partial(_gather_body, n_per_w, D),
        out_type=jax.ShapeDtypeStruct((B,), jnp.float32),
        mesh=mesh,
        scratch_types=[
            pltpu.VMEM((n_per_w,), jnp.int32),
            pltpu.VMEM((n_per_w,), jnp.int32),
            pltpu.VMEM((n_per_w,), jnp.int32),
            pltpu.VMEM((D, n_per_w), jnp.float32),
            pltpu.VMEM((D, n_per_w), jnp.float32),
            pltpu.VMEM((D, n_per_w), jnp.float32),
            pltpu.VMEM((n_per_w,), jnp.float32),
            pltpu.VMEM((n_per_w,), jnp.float32),
            pltpu.SemaphoreType.DMA,
        ],
        compiler_params=pltpu.CompilerParams(
            needs_layout_passes=False, use_tc_tiling_on_sc=False),
    )
    out = gather(x[:, 0], x[:, 1], x[:, 2], lin_in, lin_w,
                 emb_out_b.reshape(V))
    return out.reshape(B, 1)
